# initial kernel scaffold (unmeasured)
import jax
import jax.numpy as jnp
from jax import lax
from jax.experimental import pallas as pl
from jax.experimental.pallas import tpu as pltpu

N_DEV = 32


def kernel(x, w_mat):
    m, _ = x.shape
    _, n = w_mat.shape
    chunk_m = m // N_DEV

    def body(x_ref, w_ref, out_ref, comm_ref, amax_tx_ref, amax_rx_ref,
             send_sems, recv_sems, credit_sems, amax_send_sems,
             amax_recv_sems):
        my_i = lax.axis_index("i")
        left = (my_i - 1) % N_DEV
        right = (my_i + 1) % N_DEV

        barrier_sem = pltpu.get_barrier_semaphore()
        for nbr in (left, right):
            pl.semaphore_signal(barrier_sem, inc=1, device_id=(nbr,),
                                device_id_type=pl.DeviceIdType.MESH)
        pl.semaphore_wait(barrier_sem, 2)

        def partial_chunk(c):
            rows = x_ref[pl.ds(c * chunk_m, chunk_m), :]
            return jnp.dot(rows, w_ref[...],
                           preferred_element_type=jnp.float32,
                           precision=lax.Precision.HIGHEST)

        comm_ref[0] = partial_chunk((my_i - 1) % N_DEV)

        amax_local = jnp.float32(0.0)
        for h in range(N_DEV - 1):
            send_slot = h % 2
            recv_slot = (h + 1) % 2
            if h >= 1:
                pl.semaphore_wait(credit_sems.at[recv_slot], 1)
            rdma = pltpu.make_async_remote_copy(
                src_ref=comm_ref.at[send_slot],
                dst_ref=comm_ref.at[recv_slot],
                send_sem=send_sems.at[send_slot],
                recv_sem=recv_sems.at[recv_slot],
                device_id=(right,),
                device_id_type=pl.DeviceIdType.MESH,
            )
            rdma.start()
            part = partial_chunk((my_i - h - 2) % N_DEV)
            rdma.wait()
            if h < N_DEV - 2:
                comm_ref[recv_slot] = comm_ref[recv_slot] + part
                pl.semaphore_signal(credit_sems.at[send_slot], inc=1,
                                    device_id=(left,),
                                    device_id_type=pl.DeviceIdType.MESH)
            else:
                z = jnp.maximum(comm_ref[recv_slot] + part, 0.0)
                out_ref[...] = z
                amax_local = jnp.max(z)

        amax_tx_ref[...] = jnp.full((8, 128), amax_local, jnp.float32)
        sends = []
        for d in range(1, N_DEV):
            p = (my_i + d) % N_DEV
            s_rdma = pltpu.make_async_remote_copy(
                src_ref=amax_tx_ref,
                dst_ref=amax_rx_ref.at[my_i],
                send_sem=amax_send_sems.at[d - 1],
                recv_sem=amax_recv_sems.at[my_i],
                device_id=(p,),
                device_id_type=pl.DeviceIdType.MESH,
            )
            s_rdma.start()
            sends.append(s_rdma)
        for d in range(1, N_DEV):
            src = (my_i + d) % N_DEV
            recv = pltpu.make_async_remote_copy(
                src_ref=amax_tx_ref,
                dst_ref=amax_rx_ref.at[src],
                send_sem=amax_send_sems.at[0],
                recv_sem=amax_recv_sems.at[src],
                device_id=(src,),
                device_id_type=pl.DeviceIdType.MESH,
            )
            recv.wait_recv()
        for s_rdma in sends:
            s_rdma.wait_send()

        rx = amax_rx_ref[...]
        iota = lax.broadcasted_iota(jnp.int32, rx.shape, 0)
        amax_g = jnp.max(jnp.where(iota == my_i, amax_local, rx))

        scale = amax_g / 127.0
        q = jnp.clip(jnp.round(out_ref[...] / scale), -127.0, 127.0)
        out_ref[...] = q * scale

    return pl.pallas_call(
        body,
        out_shape=jax.ShapeDtypeStruct((chunk_m, n), jnp.float32),
        in_specs=[pl.BlockSpec(memory_space=pltpu.VMEM),
                  pl.BlockSpec(memory_space=pltpu.VMEM)],
        out_specs=pl.BlockSpec(memory_space=pltpu.VMEM),
        scratch_shapes=[
            pltpu.VMEM((2, chunk_m, n), jnp.float32),
            pltpu.VMEM((8, 128), jnp.float32),
            pltpu.VMEM((N_DEV, 8, 128), jnp.float32),
            pltpu.SemaphoreType.DMA((2,)),
            pltpu.SemaphoreType.DMA((2,)),
            pltpu.SemaphoreType.REGULAR((2,)),
            pltpu.SemaphoreType.DMA((N_DEV - 1,)),
            pltpu.SemaphoreType.DMA((N_DEV,)),
        ],
        compiler_params=pltpu.CompilerParams(collective_id=0),
    )(x, w_mat)


# baseline (device time: 1487613 ns/iter reference)
import jax
import jax.numpy as jnp
from jax import lax
from jax.experimental import pallas as pl
from jax.experimental.pallas import tpu as pltpu

N_DEV = 32


def kernel(x, w_mat):
    m, _ = x.shape
    _, n = w_mat.shape
    chunk_m = m // N_DEV

    def body(x_ref, w_ref, out_ref, comm_ref, amax_tx_ref, amax_rx_ref,
             send_sems, recv_sems, amax_send_sems, amax_recv_sems):
        my_i = lax.axis_index("i")
        left = (my_i - 1) % N_DEV
        right = (my_i + 1) % N_DEV

        barrier_sem = pltpu.get_barrier_semaphore()
        for nbr in (left, right):
            pl.semaphore_signal(barrier_sem, inc=1, device_id=(nbr,),
                                device_id_type=pl.DeviceIdType.MESH)
        pl.semaphore_wait(barrier_sem, 2)

        def partial_chunk(c):
            rows = x_ref[pl.ds(c * chunk_m, chunk_m), :]
            return jnp.dot(rows, w_ref[...],
                           preferred_element_type=jnp.float32,
                           precision=lax.Precision.HIGHEST)

        def hop_rdma(send_slot, recv_slot):
            return pltpu.make_async_remote_copy(
                src_ref=comm_ref.at[send_slot],
                dst_ref=comm_ref.at[recv_slot],
                send_sem=send_sems.at[send_slot],
                recv_sem=recv_sems.at[recv_slot],
                device_id=(right,),
                device_id_type=pl.DeviceIdType.MESH,
            )

        comm_ref[0] = partial_chunk((my_i - 1) % N_DEV)

        def hop_body(h, _):
            send_slot = h % 2
            recv_slot = 1 - send_slot
            rdma = hop_rdma(send_slot, recv_slot)
            rdma.start()
            part = partial_chunk((my_i - h - 2) % N_DEV)
            rdma.wait()
            comm_ref[recv_slot] = comm_ref[recv_slot] + part
            return 0

        lax.fori_loop(0, N_DEV - 2, hop_body, 0)

        rdma = hop_rdma((N_DEV - 2) % 2, 1 - (N_DEV - 2) % 2)
        rdma.start()
        part = partial_chunk(my_i)
        rdma.wait()
        z = jnp.maximum(comm_ref[1 - (N_DEV - 2) % 2] + part, 0.0)
        out_ref[...] = z
        amax_local = jnp.max(z)

        amax_tx_ref[...] = jnp.full((8, 128), amax_local, jnp.float32)
        sends = []
        for d in range(1, N_DEV):
            p = (my_i + d) % N_DEV
            s_rdma = pltpu.make_async_remote_copy(
                src_ref=amax_tx_ref,
                dst_ref=amax_rx_ref.at[my_i],
                send_sem=amax_send_sems.at[d - 1],
                recv_sem=amax_recv_sems.at[my_i],
                device_id=(p,),
                device_id_type=pl.DeviceIdType.MESH,
            )
            s_rdma.start()
            sends.append(s_rdma)
        for d in range(1, N_DEV):
            src = (my_i + d) % N_DEV
            recv = pltpu.make_async_remote_copy(
                src_ref=amax_tx_ref,
                dst_ref=amax_rx_ref.at[src],
                send_sem=amax_send_sems.at[0],
                recv_sem=amax_recv_sems.at[src],
                device_id=(src,),
                device_id_type=pl.DeviceIdType.MESH,
            )
            recv.wait_recv()
        for s_rdma in sends:
            s_rdma.wait_send()

        rx = amax_rx_ref[...]
        iota = lax.broadcasted_iota(jnp.int32, rx.shape, 0)
        amax_g = jnp.max(jnp.where(iota == my_i, amax_local, rx))

        scale = amax_g / 127.0
        q = jnp.clip(jnp.round(out_ref[...] / scale), -127.0, 127.0)
        out_ref[...] = q * scale

    return pl.pallas_call(
        body,
        out_shape=jax.ShapeDtypeStruct((chunk_m, n), jnp.float32),
        in_specs=[pl.BlockSpec(memory_space=pltpu.VMEM),
                  pl.BlockSpec(memory_space=pltpu.VMEM)],
        out_specs=pl.BlockSpec(memory_space=pltpu.VMEM),
        scratch_shapes=[
            pltpu.VMEM((2, chunk_m, n), jnp.float32),
            pltpu.VMEM((8, 128), jnp.float32),
            pltpu.VMEM((N_DEV, 8, 128), jnp.float32),
            pltpu.SemaphoreType.DMA((2,)),
            pltpu.SemaphoreType.DMA((2,)),
            pltpu.SemaphoreType.DMA((N_DEV - 1,)),
            pltpu.SemaphoreType.DMA((N_DEV,)),
        ],
        compiler_params=pltpu.CompilerParams(collective_id=0),
    )(x, w_mat)


# device time: 1478297 ns/iter; 1.0063x vs baseline; 1.0063x over previous
import jax
import jax.numpy as jnp
from jax import lax
from jax.experimental import pallas as pl
from jax.experimental.pallas import tpu as pltpu

N_DEV = 32


def kernel(x, w_mat):
    m, _ = x.shape
    _, n = w_mat.shape
    chunk_m = m // N_DEV

    half = n // 2

    def body(x_ref, w_ref, out_ref, comm_r_ref, comm_l_ref, amax_tx_ref,
             amax_rx_ref, send_r_sems, recv_r_sems, send_l_sems, recv_l_sems,
             amax_send_sems, amax_recv_sems):
        my_i = lax.axis_index("i")
        left = (my_i - 1) % N_DEV
        right = (my_i + 1) % N_DEV

        barrier_sem = pltpu.get_barrier_semaphore()
        for nbr in (left, right):
            pl.semaphore_signal(barrier_sem, inc=1, device_id=(nbr,),
                                device_id_type=pl.DeviceIdType.MESH)
        pl.semaphore_wait(barrier_sem, 2)

        def part_r(c):
            rows = x_ref[pl.ds(c * chunk_m, chunk_m), :]
            return jnp.dot(rows, w_ref[:, :half],
                           preferred_element_type=jnp.float32,
                           precision=lax.Precision.HIGHEST)

        def part_l(c):
            rows = x_ref[pl.ds(c * chunk_m, chunk_m), :]
            return jnp.dot(rows, w_ref[:, half:],
                           preferred_element_type=jnp.float32,
                           precision=lax.Precision.HIGHEST)

        def hop_rdmas(send_slot, recv_slot):
            r = pltpu.make_async_remote_copy(
                src_ref=comm_r_ref.at[send_slot],
                dst_ref=comm_r_ref.at[recv_slot],
                send_sem=send_r_sems.at[send_slot],
                recv_sem=recv_r_sems.at[recv_slot],
                device_id=(right,),
                device_id_type=pl.DeviceIdType.MESH,
            )
            l = pltpu.make_async_remote_copy(
                src_ref=comm_l_ref.at[send_slot],
                dst_ref=comm_l_ref.at[recv_slot],
                send_sem=send_l_sems.at[send_slot],
                recv_sem=recv_l_sems.at[recv_slot],
                device_id=(left,),
                device_id_type=pl.DeviceIdType.MESH,
            )
            return r, l

        comm_r_ref[0] = part_r((my_i - 1) % N_DEV)
        comm_l_ref[0] = part_l((my_i + 1) % N_DEV)

        def hop_body(h, _):
            send_slot = h % 4
            recv_slot = (h + 1) % 4
            r, l = hop_rdmas(send_slot, recv_slot)
            r.start()
            l.start()
            pr = part_r((my_i - h - 2) % N_DEV)
            pl_ = part_l((my_i + h + 2) % N_DEV)
            r.wait()
            comm_r_ref[recv_slot] = comm_r_ref[recv_slot] + pr
            l.wait()
            comm_l_ref[recv_slot] = comm_l_ref[recv_slot] + pl_
            return 0

        lax.fori_loop(0, N_DEV - 2, hop_body, 0)

        fs = (N_DEV - 2) % 4
        fr = (N_DEV - 1) % 4
        r, l = hop_rdmas(fs, fr)
        r.start()
        l.start()
        pr = part_r(my_i)
        pl_ = part_l(my_i)
        r.wait()
        zr = jnp.maximum(comm_r_ref[fr] + pr, 0.0)
        out_ref[:, :half] = zr
        l.wait()
        zl = jnp.maximum(comm_l_ref[fr] + pl_, 0.0)
        out_ref[:, half:] = zl
        amax_local = jnp.maximum(jnp.max(zr), jnp.max(zl))

        amax_tx_ref[...] = jnp.full((8, 128), amax_local, jnp.float32)
        sends = []
        for d in range(1, N_DEV):
            p = (my_i + d) % N_DEV
            s_rdma = pltpu.make_async_remote_copy(
                src_ref=amax_tx_ref,
                dst_ref=amax_rx_ref.at[my_i],
                send_sem=amax_send_sems.at[d - 1],
                recv_sem=amax_recv_sems.at[my_i],
                device_id=(p,),
                device_id_type=pl.DeviceIdType.MESH,
            )
            s_rdma.start()
            sends.append(s_rdma)
        for d in range(1, N_DEV):
            src = (my_i + d) % N_DEV
            recv = pltpu.make_async_remote_copy(
                src_ref=amax_tx_ref,
                dst_ref=amax_rx_ref.at[src],
                send_sem=amax_send_sems.at[0],
                recv_sem=amax_recv_sems.at[src],
                device_id=(src,),
                device_id_type=pl.DeviceIdType.MESH,
            )
            recv.wait_recv()
        for s_rdma in sends:
            s_rdma.wait_send()

        rx = amax_rx_ref[...]
        iota = lax.broadcasted_iota(jnp.int32, rx.shape, 0)
        amax_g = jnp.max(jnp.where(iota == my_i, amax_local, rx))

        scale = amax_g / 127.0
        q = jnp.clip(jnp.round(out_ref[...] / scale), -127.0, 127.0)
        out_ref[...] = q * scale

    return pl.pallas_call(
        body,
        out_shape=jax.ShapeDtypeStruct((chunk_m, n), jnp.float32),
        in_specs=[pl.BlockSpec(memory_space=pltpu.VMEM),
                  pl.BlockSpec(memory_space=pltpu.VMEM)],
        out_specs=pl.BlockSpec(memory_space=pltpu.VMEM),
        scratch_shapes=[
            pltpu.VMEM((4, chunk_m, n // 2), jnp.float32),
            pltpu.VMEM((4, chunk_m, n // 2), jnp.float32),
            pltpu.VMEM((8, 128), jnp.float32),
            pltpu.VMEM((N_DEV, 8, 128), jnp.float32),
            pltpu.SemaphoreType.DMA((4,)),
            pltpu.SemaphoreType.DMA((4,)),
            pltpu.SemaphoreType.DMA((4,)),
            pltpu.SemaphoreType.DMA((4,)),
            pltpu.SemaphoreType.DMA((N_DEV - 1,)),
            pltpu.SemaphoreType.DMA((N_DEV,)),
        ],
        compiler_params=pltpu.CompilerParams(collective_id=0),
    )(x, w_mat)


# device time: 1475194 ns/iter; 1.0084x vs baseline; 1.0021x over previous
import jax
import jax.numpy as jnp
from jax import lax
from jax.experimental import pallas as pl
from jax.experimental.pallas import tpu as pltpu

N_DEV = 32


def kernel(x, w_mat):
    m, _ = x.shape
    _, n = w_mat.shape
    chunk_m = m // N_DEV

    half = n // 2

    def body(x_ref, w_ref, out_ref, comm_r_ref, comm_l_ref, amax_tx_ref,
             amax_rx_ref, send_r_sems, recv_r_sems, send_l_sems, recv_l_sems,
             amax_send_sems, amax_recv_sems):
        my_i = lax.axis_index("i")
        left = (my_i - 1) % N_DEV
        right = (my_i + 1) % N_DEV

        barrier_sem = pltpu.get_barrier_semaphore()
        for nbr in (left, right):
            pl.semaphore_signal(barrier_sem, inc=1, device_id=(nbr,),
                                device_id_type=pl.DeviceIdType.MESH)
        pl.semaphore_wait(barrier_sem, 2)

        def part_r(c):
            rows = x_ref[pl.ds(c * chunk_m, chunk_m), :]
            return jnp.dot(rows, w_ref[:, :half],
                           preferred_element_type=jnp.float32,
                           precision=lax.Precision.DEFAULT)

        def part_l(c):
            rows = x_ref[pl.ds(c * chunk_m, chunk_m), :]
            return jnp.dot(rows, w_ref[:, half:],
                           preferred_element_type=jnp.float32,
                           precision=lax.Precision.DEFAULT)

        def hop_rdmas(send_slot, recv_slot):
            r = pltpu.make_async_remote_copy(
                src_ref=comm_r_ref.at[send_slot],
                dst_ref=comm_r_ref.at[recv_slot],
                send_sem=send_r_sems.at[send_slot],
                recv_sem=recv_r_sems.at[recv_slot],
                device_id=(right,),
                device_id_type=pl.DeviceIdType.MESH,
            )
            l = pltpu.make_async_remote_copy(
                src_ref=comm_l_ref.at[send_slot],
                dst_ref=comm_l_ref.at[recv_slot],
                send_sem=send_l_sems.at[send_slot],
                recv_sem=recv_l_sems.at[recv_slot],
                device_id=(left,),
                device_id_type=pl.DeviceIdType.MESH,
            )
            return r, l

        comm_r_ref[0] = part_r((my_i - 1) % N_DEV)
        comm_l_ref[0] = part_l((my_i + 1) % N_DEV)

        def hop_body(h, _):
            send_slot = h % 4
            recv_slot = (h + 1) % 4
            r, l = hop_rdmas(send_slot, recv_slot)
            r.start()
            l.start()
            pr = part_r((my_i - h - 2) % N_DEV)
            pl_ = part_l((my_i + h + 2) % N_DEV)
            r.wait()
            comm_r_ref[recv_slot] = comm_r_ref[recv_slot] + pr
            l.wait()
            comm_l_ref[recv_slot] = comm_l_ref[recv_slot] + pl_
            return 0

        lax.fori_loop(0, N_DEV - 2, hop_body, 0)

        fs = (N_DEV - 2) % 4
        fr = (N_DEV - 1) % 4
        r, l = hop_rdmas(fs, fr)
        r.start()
        l.start()
        pr = part_r(my_i)
        pl_ = part_l(my_i)
        r.wait()
        zr = jnp.maximum(comm_r_ref[fr] + pr, 0.0)
        out_ref[:, :half] = zr
        l.wait()
        zl = jnp.maximum(comm_l_ref[fr] + pl_, 0.0)
        out_ref[:, half:] = zl
        amax_local = jnp.maximum(jnp.max(zr), jnp.max(zl))

        amax_tx_ref[...] = jnp.full((8, 128), amax_local, jnp.float32)
        sends = []
        for d in range(1, N_DEV):
            p = (my_i + d) % N_DEV
            s_rdma = pltpu.make_async_remote_copy(
                src_ref=amax_tx_ref,
                dst_ref=amax_rx_ref.at[my_i],
                send_sem=amax_send_sems.at[d - 1],
                recv_sem=amax_recv_sems.at[my_i],
                device_id=(p,),
                device_id_type=pl.DeviceIdType.MESH,
            )
            s_rdma.start()
            sends.append(s_rdma)
        for d in range(1, N_DEV):
            src = (my_i + d) % N_DEV
            recv = pltpu.make_async_remote_copy(
                src_ref=amax_tx_ref,
                dst_ref=amax_rx_ref.at[src],
                send_sem=amax_send_sems.at[0],
                recv_sem=amax_recv_sems.at[src],
                device_id=(src,),
                device_id_type=pl.DeviceIdType.MESH,
            )
            recv.wait_recv()
        for s_rdma in sends:
            s_rdma.wait_send()

        rx = amax_rx_ref[...]
        iota = lax.broadcasted_iota(jnp.int32, rx.shape, 0)
        amax_g = jnp.max(jnp.where(iota == my_i, amax_local, rx))

        scale = amax_g / 127.0
        q = jnp.clip(jnp.round(out_ref[...] / scale), -127.0, 127.0)
        out_ref[...] = q * scale

    return pl.pallas_call(
        body,
        out_shape=jax.ShapeDtypeStruct((chunk_m, n), jnp.float32),
        in_specs=[pl.BlockSpec(memory_space=pltpu.VMEM),
                  pl.BlockSpec(memory_space=pltpu.VMEM)],
        out_specs=pl.BlockSpec(memory_space=pltpu.VMEM),
        scratch_shapes=[
            pltpu.VMEM((4, chunk_m, n // 2), jnp.float32),
            pltpu.VMEM((4, chunk_m, n // 2), jnp.float32),
            pltpu.VMEM((8, 128), jnp.float32),
            pltpu.VMEM((N_DEV, 8, 128), jnp.float32),
            pltpu.SemaphoreType.DMA((4,)),
            pltpu.SemaphoreType.DMA((4,)),
            pltpu.SemaphoreType.DMA((4,)),
            pltpu.SemaphoreType.DMA((4,)),
            pltpu.SemaphoreType.DMA((N_DEV - 1,)),
            pltpu.SemaphoreType.DMA((N_DEV,)),
        ],
        compiler_params=pltpu.CompilerParams(collective_id=0),
    )(x, w_mat)


# device time: 786317 ns/iter; 1.8919x vs baseline; 1.8761x over previous
import jax
import jax.numpy as jnp
from jax import lax
from jax.experimental import pallas as pl
from jax.experimental.pallas import tpu as pltpu

N_DEV = 32

PERM = [0, 1, 2, 3, 4, 5, 6, 7, 15, 14, 13, 12, 11, 10, 18, 19,
        20, 21, 22, 23, 31, 30, 29, 28, 27, 26, 25, 24, 16, 17, 9, 8]
INV_PERM = [0] * N_DEV
for _k, _p in enumerate(PERM):
    INV_PERM[_p] = _k


def kernel(x, w_mat):
    m, _ = x.shape
    _, n = w_mat.shape
    chunk_m = m // N_DEV
    half = n // 2

    perm_arr = jnp.asarray(PERM, jnp.int32)
    inv_arr = jnp.asarray(INV_PERM, jnp.int32)

    def body(x_ref, w_ref, perm_ref, inv_ref, out_ref, comm_r_ref,
             comm_l_ref, amax_tx_ref, amax_rx_ref, send_r_sems, recv_r_sems,
             send_l_sems, recv_l_sems, amax_send_sems, amax_recv_sems):
        my_i = lax.axis_index("i")
        rank = inv_ref[my_i]
        left = perm_ref[(rank - 1) % N_DEV]
        right = perm_ref[(rank + 1) % N_DEV]

        barrier_sem = pltpu.get_barrier_semaphore()
        for nbr in (left, right):
            pl.semaphore_signal(barrier_sem, inc=1, device_id=(nbr,),
                                device_id_type=pl.DeviceIdType.MESH)
        pl.semaphore_wait(barrier_sem, 2)

        def part_r(k):
            c = perm_ref[k % N_DEV]
            rows = x_ref[pl.ds(c * chunk_m, chunk_m), :]
            return jnp.dot(rows, w_ref[:, :half],
                           preferred_element_type=jnp.float32,
                           precision=lax.Precision.HIGHEST)

        def part_l(k):
            c = perm_ref[k % N_DEV]
            rows = x_ref[pl.ds(c * chunk_m, chunk_m), :]
            return jnp.dot(rows, w_ref[:, half:],
                           preferred_element_type=jnp.float32,
                           precision=lax.Precision.HIGHEST)

        def hop_rdmas(send_slot, recv_slot):
            r = pltpu.make_async_remote_copy(
                src_ref=comm_r_ref.at[send_slot],
                dst_ref=comm_r_ref.at[recv_slot],
                send_sem=send_r_sems.at[send_slot],
                recv_sem=recv_r_sems.at[recv_slot],
                device_id=(right,),
                device_id_type=pl.DeviceIdType.MESH,
            )
            l = pltpu.make_async_remote_copy(
                src_ref=comm_l_ref.at[send_slot],
                dst_ref=comm_l_ref.at[recv_slot],
                send_sem=send_l_sems.at[send_slot],
                recv_sem=recv_l_sems.at[recv_slot],
                device_id=(left,),
                device_id_type=pl.DeviceIdType.MESH,
            )
            return r, l

        comm_r_ref[0] = part_r(rank - 1 + N_DEV)
        comm_l_ref[0] = part_l(rank + 1)

        def hop_body(h, _):
            send_slot = h % 4
            recv_slot = (h + 1) % 4
            r, l = hop_rdmas(send_slot, recv_slot)
            r.start()
            l.start()
            pr = part_r(rank - h - 2 + 2 * N_DEV)
            pl_ = part_l(rank + h + 2)
            r.wait()
            comm_r_ref[recv_slot] = comm_r_ref[recv_slot] + pr
            l.wait()
            comm_l_ref[recv_slot] = comm_l_ref[recv_slot] + pl_
            return 0

        lax.fori_loop(0, N_DEV - 2, hop_body, 0)

        fs = (N_DEV - 2) % 4
        fr = (N_DEV - 1) % 4
        r, l = hop_rdmas(fs, fr)
        r.start()
        l.start()
        pr = part_r(rank)
        pl_ = part_l(rank)
        r.wait()
        zr = jnp.maximum(comm_r_ref[fr] + pr, 0.0)
        out_ref[:, :half] = zr
        l.wait()
        zl = jnp.maximum(comm_l_ref[fr] + pl_, 0.0)
        out_ref[:, half:] = zl
        amax_local = jnp.maximum(jnp.max(zr), jnp.max(zl))

        amax_tx_ref[...] = jnp.full((8, 128), amax_local, jnp.float32)
        sends = []
        for d in range(1, N_DEV):
            p = (my_i + d) % N_DEV
            s_rdma = pltpu.make_async_remote_copy(
                src_ref=amax_tx_ref,
                dst_ref=amax_rx_ref.at[my_i],
                send_sem=amax_send_sems.at[d - 1],
                recv_sem=amax_recv_sems.at[my_i],
                device_id=(p,),
                device_id_type=pl.DeviceIdType.MESH,
            )
            s_rdma.start()
            sends.append(s_rdma)
        for d in range(1, N_DEV):
            src = (my_i + d) % N_DEV
            recv = pltpu.make_async_remote_copy(
                src_ref=amax_tx_ref,
                dst_ref=amax_rx_ref.at[src],
                send_sem=amax_send_sems.at[0],
                recv_sem=amax_recv_sems.at[src],
                device_id=(src,),
                device_id_type=pl.DeviceIdType.MESH,
            )
            recv.wait_recv()
        for s_rdma in sends:
            s_rdma.wait_send()

        rx = amax_rx_ref[...]
        iota = lax.broadcasted_iota(jnp.int32, rx.shape, 0)
        amax_g = jnp.max(jnp.where(iota == my_i, amax_local, rx))

        scale = amax_g / 127.0
        q = jnp.clip(jnp.round(out_ref[...] / scale), -127.0, 127.0)
        out_ref[...] = q * scale

    return pl.pallas_call(
        body,
        out_shape=jax.ShapeDtypeStruct((chunk_m, n), jnp.float32),
        in_specs=[pl.BlockSpec(memory_space=pltpu.VMEM),
                  pl.BlockSpec(memory_space=pltpu.VMEM),
                  pl.BlockSpec(memory_space=pltpu.SMEM),
                  pl.BlockSpec(memory_space=pltpu.SMEM)],
        out_specs=pl.BlockSpec(memory_space=pltpu.VMEM),
        scratch_shapes=[
            pltpu.VMEM((4, chunk_m, n // 2), jnp.float32),
            pltpu.VMEM((4, chunk_m, n // 2), jnp.float32),
            pltpu.VMEM((8, 128), jnp.float32),
            pltpu.VMEM((N_DEV, 8, 128), jnp.float32),
            pltpu.SemaphoreType.DMA((4,)),
            pltpu.SemaphoreType.DMA((4,)),
            pltpu.SemaphoreType.DMA((4,)),
            pltpu.SemaphoreType.DMA((4,)),
            pltpu.SemaphoreType.DMA((N_DEV - 1,)),
            pltpu.SemaphoreType.DMA((N_DEV,)),
        ],
        compiler_params=pltpu.CompilerParams(collective_id=0),
    )(x, w_mat, perm_arr, inv_arr)


# device time: 779756 ns/iter; 1.9078x vs baseline; 1.0084x over previous
import jax
import jax.numpy as jnp
from jax import lax
from jax.experimental import pallas as pl
from jax.experimental.pallas import tpu as pltpu

N_DEV = 32

PERM = [0, 1, 2, 3, 4, 5, 6, 7, 15, 14, 13, 12, 11, 10, 18, 19,
        20, 21, 22, 23, 31, 30, 29, 28, 27, 26, 25, 24, 16, 17, 9, 8]
INV_PERM = [0] * N_DEV
for _k, _p in enumerate(PERM):
    INV_PERM[_p] = _k


def kernel(x, w_mat):
    m, _ = x.shape
    _, n = w_mat.shape
    chunk_m = m // N_DEV
    half = n // 2

    perm_arr = jnp.asarray(PERM, jnp.int32)
    inv_arr = jnp.asarray(INV_PERM, jnp.int32)

    def body(x_ref, w_ref, perm_ref, inv_ref, out_ref, comm_r_ref,
             comm_l_ref, amax_tx_ref, amax_rx_ref, send_r_sems, recv_r_sems,
             send_l_sems, recv_l_sems, amax_send_sems, amax_recv_sems):
        my_i = lax.axis_index("i")
        rank = inv_ref[my_i]
        left = perm_ref[(rank - 1) % N_DEV]
        right = perm_ref[(rank + 1) % N_DEV]

        barrier_sem = pltpu.get_barrier_semaphore()
        for nbr in (left, right):
            pl.semaphore_signal(barrier_sem, inc=1, device_id=(nbr,),
                                device_id_type=pl.DeviceIdType.MESH)
        pl.semaphore_wait(barrier_sem, 2)

        def part_r(k):
            c = perm_ref[k % N_DEV]
            rows = x_ref[pl.ds(c * chunk_m, chunk_m), :]
            return jnp.dot(rows, w_ref[:, :half],
                           preferred_element_type=jnp.float32,
                           precision=lax.Precision.HIGHEST)

        def part_l(k):
            c = perm_ref[k % N_DEV]
            rows = x_ref[pl.ds(c * chunk_m, chunk_m), :]
            return jnp.dot(rows, w_ref[:, half:],
                           preferred_element_type=jnp.float32,
                           precision=lax.Precision.HIGHEST)

        def hop_rdmas(send_slot, recv_slot):
            r = pltpu.make_async_remote_copy(
                src_ref=comm_r_ref.at[send_slot],
                dst_ref=comm_r_ref.at[recv_slot],
                send_sem=send_r_sems.at[send_slot],
                recv_sem=recv_r_sems.at[recv_slot],
                device_id=(right,),
                device_id_type=pl.DeviceIdType.MESH,
            )
            l = pltpu.make_async_remote_copy(
                src_ref=comm_l_ref.at[send_slot],
                dst_ref=comm_l_ref.at[recv_slot],
                send_sem=send_l_sems.at[send_slot],
                recv_sem=recv_l_sems.at[recv_slot],
                device_id=(left,),
                device_id_type=pl.DeviceIdType.MESH,
            )
            return r, l

        comm_r_ref[0] = part_r(rank - 1 + N_DEV)
        comm_l_ref[0] = part_l(rank + 1)

        r0, l0 = hop_rdmas(0, 1)
        r0.start()
        l0.start()

        def hop_body(h, _):
            recv_slot = (h + 1) % 4
            pr = part_r(rank - h - 2 + 2 * N_DEV)
            pl_ = part_l(rank + h + 2)
            rw, lw = hop_rdmas(h % 4, recv_slot)
            rn, ln = hop_rdmas(recv_slot, (h + 2) % 4)
            rw.wait()
            comm_r_ref[recv_slot] = comm_r_ref[recv_slot] + pr
            rn.start()
            lw.wait()
            comm_l_ref[recv_slot] = comm_l_ref[recv_slot] + pl_
            ln.start()
            return 0

        lax.fori_loop(0, N_DEV - 2, hop_body, 0)

        fs = (N_DEV - 2) % 4
        fr = (N_DEV - 1) % 4
        pr = part_r(rank)
        pl_ = part_l(rank)
        rw, lw = hop_rdmas(fs, fr)
        rw.wait()
        zr = jnp.maximum(comm_r_ref[fr] + pr, 0.0)
        out_ref[:, :half] = zr
        lw.wait()
        zl = jnp.maximum(comm_l_ref[fr] + pl_, 0.0)
        out_ref[:, half:] = zl
        amax_local = jnp.maximum(jnp.max(zr), jnp.max(zl))

        amax_tx_ref[...] = jnp.full((8, 128), amax_local, jnp.float32)
        sends = []
        for d in range(1, N_DEV):
            p = (my_i + d) % N_DEV
            s_rdma = pltpu.make_async_remote_copy(
                src_ref=amax_tx_ref,
                dst_ref=amax_rx_ref.at[my_i],
                send_sem=amax_send_sems.at[d - 1],
                recv_sem=amax_recv_sems.at[my_i],
                device_id=(p,),
                device_id_type=pl.DeviceIdType.MESH,
            )
            s_rdma.start()
            sends.append(s_rdma)
        for d in range(1, N_DEV):
            src = (my_i + d) % N_DEV
            recv = pltpu.make_async_remote_copy(
                src_ref=amax_tx_ref,
                dst_ref=amax_rx_ref.at[src],
                send_sem=amax_send_sems.at[0],
                recv_sem=amax_recv_sems.at[src],
                device_id=(src,),
                device_id_type=pl.DeviceIdType.MESH,
            )
            recv.wait_recv()
        for s_rdma in sends:
            s_rdma.wait_send()

        rx = amax_rx_ref[...]
        iota = lax.broadcasted_iota(jnp.int32, rx.shape, 0)
        amax_g = jnp.max(jnp.where(iota == my_i, amax_local, rx))

        scale = amax_g / 127.0
        q = jnp.clip(jnp.round(out_ref[...] / scale), -127.0, 127.0)
        out_ref[...] = q * scale

    return pl.pallas_call(
        body,
        out_shape=jax.ShapeDtypeStruct((chunk_m, n), jnp.float32),
        in_specs=[pl.BlockSpec(memory_space=pltpu.VMEM),
                  pl.BlockSpec(memory_space=pltpu.VMEM),
                  pl.BlockSpec(memory_space=pltpu.SMEM),
                  pl.BlockSpec(memory_space=pltpu.SMEM)],
        out_specs=pl.BlockSpec(memory_space=pltpu.VMEM),
        scratch_shapes=[
            pltpu.VMEM((4, chunk_m, n // 2), jnp.float32),
            pltpu.VMEM((4, chunk_m, n // 2), jnp.float32),
            pltpu.VMEM((8, 128), jnp.float32),
            pltpu.VMEM((N_DEV, 8, 128), jnp.float32),
            pltpu.SemaphoreType.DMA((4,)),
            pltpu.SemaphoreType.DMA((4,)),
            pltpu.SemaphoreType.DMA((4,)),
            pltpu.SemaphoreType.DMA((4,)),
            pltpu.SemaphoreType.DMA((N_DEV - 1,)),
            pltpu.SemaphoreType.DMA((N_DEV,)),
        ],
        compiler_params=pltpu.CompilerParams(collective_id=0),
    )(x, w_mat, perm_arr, inv_arr)
